# Initial kernel scaffold; baseline (speedup 1.0000x reference)
import jax
import jax.numpy as jnp
from jax import lax
from jax.experimental import pallas as pl
from jax.experimental.pallas import tpu as pltpu

N = 2048
DIM = 512
NS = 256
C = 16
KP1 = 12
BR = 256
NB = N // BR
BIGF = 3.0e38
SHIFT0 = 1.0e6
SHIFT = 1.0


def _body(fs_ref, ys_ref, fq_ref, out_ref,
          dist_ref, eye_ref, q2_ref, q2t_ref, sig_ref, sigt_ref,
          a_ref, lam_ref, y_ref, rs_ref, cs_ref, coef_ref):
    def eye_blk(bi, _):
        rows = lax.broadcasted_iota(jnp.int32, (BR, N), 0) + bi * BR
        cols = lax.broadcasted_iota(jnp.int32, (BR, N), 1)
        eye_ref[pl.ds(bi * BR, BR), :] = (rows == cols).astype(jnp.float32)
        fq = fq_ref[pl.ds(bi * BR, BR), :]
        q2_ref[pl.ds(bi * BR, BR), :] = jnp.sum(fq * fq, axis=1, keepdims=True)
        return 0
    lax.fori_loop(0, NB, eye_blk, 0)

    eye = eye_ref[...]
    q2t_ref[...] = lax.dot_general(
        q2_ref[...], eye, (((0,), (0,)), ((), ())),
        preferred_element_type=jnp.float32)

    cls_iota = lax.broadcasted_iota(jnp.int32, (C, NS), 0)
    onehot = (cls_iota == ys_ref[...]).astype(jnp.float32)
    counts = jnp.sum(onehot, axis=1, keepdims=True)
    protos = lax.dot_general(
        onehot, fs_ref[...], (((1,), (0,)), ((), ())),
        preferred_element_type=jnp.float32) / counts

    p2 = jnp.sum(protos * protos, axis=1, keepdims=True)
    eye16 = (lax.broadcasted_iota(jnp.int32, (C, C), 0)
             == lax.broadcasted_iota(jnp.int32, (C, C), 1)).astype(jnp.float32)
    p2t = lax.dot_general(p2, eye16, (((0,), (0,)), ((), ())),
                          preferred_element_type=jnp.float32)
    pq = lax.dot_general(fq_ref[...], protos, (((1,), (1,)), ((), ())),
                         preferred_element_type=jnp.float32)
    d2p = jnp.maximum(q2_ref[...] + p2t - 2.0 * pq, 0.0)
    distp = jnp.sqrt(d2p + 1e-12)
    a_ref[...] = distp * distp
    d = jnp.min(distp, axis=1, keepdims=True)

    db = lax.bitcast_convert_type(d, jnp.int32)
    def med_step(_, c):
        lo, hi = c
        mid = lo + (hi - lo) // 2
        cnt = jnp.sum((db <= mid).astype(jnp.int32))
        take = cnt >= (N // 2)
        return jnp.where(take, lo, mid + 1), jnp.where(take, mid, hi)
    lo, _hi = lax.fori_loop(0, 31, med_step, (jnp.int32(0), jnp.int32(2**31 - 1)))
    med = lax.bitcast_convert_type(lo, jnp.float32)
    lam_ref[...] = jnp.exp(-(d * d) / (2.0 * med * med + 1e-8))

    na = -a_ref[...]
    mx = jnp.max(na, axis=1, keepdims=True)
    e = jnp.exp(na - mx)
    y_ref[...] = e / jnp.sum(e, axis=1, keepdims=True)

    def dist_blk(bi, _):
        fq = fq_ref[pl.ds(bi * BR, BR), :]
        g = lax.dot_general(fq, fq_ref[...], (((1,), (1,)), ((), ())),
                            preferred_element_type=jnp.float32)
        d2 = jnp.maximum(q2_ref[pl.ds(bi * BR, BR), :] + q2t_ref[...] - 2.0 * g, 0.0)
        dist_ref[pl.ds(bi * BR, BR), :] = jnp.sqrt(d2 + 1e-12)
        return 0
    lax.fori_loop(0, NB, dist_blk, 0)

    def ext_step(t, _):
        j = t // NB
        bi = t % NB
        shift = jnp.where(j == 0, SHIFT0, SHIFT)
        blk = dist_ref[pl.ds(bi * BR, BR), :]
        masked = jnp.where(blk < 0.0, BIGF, blk)
        m = jnp.min(masked, axis=1, keepdims=True)
        cols = lax.broadcasted_iota(jnp.int32, (BR, N), 1)
        am = jnp.min(jnp.where(masked == m, cols, N), axis=1, keepdims=True)
        chosen = cols == am
        dist_ref[pl.ds(bi * BR, BR), :] = jnp.where(chosen, -blk - shift, blk)

        @pl.when(j == KP1 - 1)
        def _():
            sig_ref[pl.ds(bi * BR, BR), :] = m + 1e-8
        return 0
    lax.fori_loop(0, KP1 * NB, ext_step, 0)

    sigt_ref[...] = lax.dot_general(sig_ref[...], eye, (((0,), (0,)), ((), ())),
                                    preferred_element_type=jnp.float32)

    cs_ref[...] = jnp.zeros((1, N), jnp.float32)
    def h_blk(bi, _):
        blk = dist_ref[pl.ds(bi * BR, BR), :]
        is_nbr = (blk < 0.0) & (blk > -1e5)
        dval = -blk - SHIFT
        sig = sig_ref[pl.ds(bi * BR, BR), :]
        h = jnp.where(is_nbr, 0.5 * jnp.exp(-dval / (sig * sigt_ref[...])), 0.0)
        dist_ref[pl.ds(bi * BR, BR), :] = h
        rs_ref[pl.ds(bi * BR, BR), :] = jnp.sum(h, axis=1, keepdims=True)
        cs_ref[...] += jnp.sum(h, axis=0, keepdims=True)
        return 0
    lax.fori_loop(0, NB, h_blk, 0)

    cs_col = lax.dot_general(eye, cs_ref[...], (((1,), (1,)), ((), ())),
                             preferred_element_type=jnp.float32)
    d_inv = 1.0 / (rs_ref[...] + cs_col + 1e-8)
    coef_ref[...] = lam_ref[...] * d_inv

    neg_a = -a_ref[...]
    coef = coef_ref[...]

    def lp_cond(c):
        i, stop = c
        return (i < 50) & jnp.logical_not(stop)

    def lp_body(c):
        i, stop = c
        y_old = y_ref[...]
        h = dist_ref[...]
        z = lax.dot_general(h, y_old, (((1,), (0,)), ((), ())),
                            preferred_element_type=jnp.float32)
        z = z + lax.dot_general(h, y_old, (((0,), (0,)), ((), ())),
                                preferred_element_type=jnp.float32)
        logits = neg_a + coef * z
        mxl = jnp.max(logits, axis=1, keepdims=True)
        el = jnp.exp(logits - mxl)
        y_new = el / jnp.sum(el, axis=1, keepdims=True)
        converged = jnp.max(jnp.abs(y_new - y_old)) < 1e-4

        @pl.when(jnp.logical_not(converged))
        def _():
            y_ref[...] = y_new
        return i + 1, converged

    lax.while_loop(lp_cond, lp_body, (jnp.int32(0), jnp.bool_(False)))

    y = y_ref[...]
    mxy = jnp.max(y, axis=1, keepdims=True)
    cols = lax.broadcasted_iota(jnp.int32, (N, C), 1)
    out_ref[...] = jnp.min(jnp.where(y == mxy, cols, C), axis=1, keepdims=True)


def kernel(feat_s, y_s, feat_q):
    ys2d = y_s.astype(jnp.int32).reshape(1, NS)
    out = pl.pallas_call(
        _body,
        out_shape=jax.ShapeDtypeStruct((N, 1), jnp.int32),
        scratch_shapes=[
            pltpu.VMEM((N, N), jnp.float32),
            pltpu.VMEM((N, N), jnp.float32),
            pltpu.VMEM((N, 1), jnp.float32),
            pltpu.VMEM((1, N), jnp.float32),
            pltpu.VMEM((N, 1), jnp.float32),
            pltpu.VMEM((1, N), jnp.float32),
            pltpu.VMEM((N, C), jnp.float32),
            pltpu.VMEM((N, 1), jnp.float32),
            pltpu.VMEM((N, C), jnp.float32),
            pltpu.VMEM((N, 1), jnp.float32),
            pltpu.VMEM((1, N), jnp.float32),
            pltpu.VMEM((N, 1), jnp.float32),
        ],
    )(feat_s, ys2d, feat_q)
    return out.reshape(N)


# speedup vs baseline: 10.2367x; 10.2367x over previous
import jax
import jax.numpy as jnp
from jax import lax
from jax.experimental import pallas as pl
from jax.experimental.pallas import tpu as pltpu

N = 2048
DIM = 512
NS = 256
C = 16
KP1 = 12
BR = 256
NB = N // BR
BIGF = 3.0e38


def _body(fs_ref, ys_ref, fq_ref, out_ref,
          dist_ref, wv_ref, q2_ref, q2t_ref, sig_ref, sigt_ref,
          a_ref, lam_ref, y_ref, coef_ref):
    eye_b = (lax.broadcasted_iota(jnp.int32, (BR, BR), 0)
             == lax.broadcasted_iota(jnp.int32, (BR, BR), 1)).astype(jnp.float32)

    def col2row(col_ref, row_ref):
        def blk(bi, _):
            row_ref[:, pl.ds(bi * BR, BR)] = lax.dot_general(
                col_ref[pl.ds(bi * BR, BR), :], eye_b, (((0,), (0,)), ((), ())),
                preferred_element_type=jnp.float32, precision=lax.Precision.HIGHEST)
            return 0
        lax.fori_loop(0, NB, blk, 0)

    def q2_blk(bi, _):
        fq = fq_ref[pl.ds(bi * BR, BR), :]
        q2_ref[pl.ds(bi * BR, BR), :] = jnp.sum(fq * fq, axis=1, keepdims=True)
        wv_ref[pl.ds(bi * BR, BR), :] = jnp.zeros((BR, N), jnp.float32)
        return 0
    lax.fori_loop(0, NB, q2_blk, 0)
    col2row(q2_ref, q2t_ref)

    cls_iota = lax.broadcasted_iota(jnp.int32, (C, NS), 0)
    onehot = (cls_iota == ys_ref[...]).astype(jnp.float32)
    counts = jnp.sum(onehot, axis=1, keepdims=True)
    protos = lax.dot_general(
        onehot, fs_ref[...], (((1,), (0,)), ((), ())),
        preferred_element_type=jnp.float32,
        precision=lax.Precision.HIGHEST) / counts

    p2 = jnp.sum(protos * protos, axis=1, keepdims=True)
    eye16 = (lax.broadcasted_iota(jnp.int32, (C, C), 0)
             == lax.broadcasted_iota(jnp.int32, (C, C), 1)).astype(jnp.float32)
    p2t = lax.dot_general(p2, eye16, (((0,), (0,)), ((), ())),
                          preferred_element_type=jnp.float32,
                          precision=lax.Precision.HIGHEST)
    pq = lax.dot_general(fq_ref[...], protos, (((1,), (1,)), ((), ())),
                         preferred_element_type=jnp.float32)
    d2p = jnp.maximum(q2_ref[...] + p2t - 2.0 * pq, 0.0)
    distp = jnp.sqrt(d2p + 1e-12)
    a_ref[...] = distp * distp
    d = jnp.min(distp, axis=1, keepdims=True)

    db = lax.bitcast_convert_type(d, jnp.int32)
    def med_step(_, c):
        lo, hi = c
        mid = lo + (hi - lo) // 2
        cnt = jnp.sum((db <= mid).astype(jnp.int32))
        take = cnt >= (N // 2)
        return jnp.where(take, lo, mid + 1), jnp.where(take, mid, hi)
    lo, _hi = lax.fori_loop(0, 31, med_step, (jnp.int32(0), jnp.int32(2**31 - 1)))
    med = lax.bitcast_convert_type(lo, jnp.float32)
    lam_ref[...] = jnp.exp(-(d * d) / (2.0 * med * med + 1e-8))

    na = -a_ref[...]
    mx = jnp.max(na, axis=1, keepdims=True)
    e = jnp.exp(na - mx)
    y_ref[...] = e / jnp.sum(e, axis=1, keepdims=True)

    def dist_blk(bi, _):
        fq = fq_ref[pl.ds(bi * BR, BR), :]
        g = lax.dot_general(fq, fq_ref[...], (((1,), (1,)), ((), ())),
                            preferred_element_type=jnp.float32)
        d2 = jnp.maximum(q2_ref[pl.ds(bi * BR, BR), :] + q2t_ref[...] - 2.0 * g, 0.0)
        dist_ref[pl.ds(bi * BR, BR), :] = jnp.sqrt(d2 + 1e-12)
        return 0
    lax.fori_loop(0, NB, dist_blk, 0)

    def ext_step(t, _):
        j = t // NB
        bi = t % NB
        sl = pl.ds(bi * BR, BR)
        blk = dist_ref[sl, :]
        m = jnp.min(blk, axis=1, keepdims=True)
        cols = lax.broadcasted_iota(jnp.int32, (BR, N), 1)
        am = jnp.min(jnp.where(blk == m, cols, N), axis=1, keepdims=True)
        chosen = cols == am
        dist_ref[sl, :] = jnp.where(chosen, BIGF, blk)

        @pl.when(j > 0)
        def _():
            wv_ref[sl, :] = jnp.where(chosen, m, wv_ref[sl, :])

        @pl.when(j == KP1 - 1)
        def _():
            sig_ref[sl, :] = m + 1e-8
        return 0
    lax.fori_loop(0, KP1 * NB, ext_step, 0)

    col2row(sig_ref, sigt_ref)

    def wexp_blk(bi, _):
        sl = pl.ds(bi * BR, BR)
        wv = wv_ref[sl, :]
        sig = sig_ref[sl, :]
        wv_ref[sl, :] = jnp.where(wv > 0.0,
                                  jnp.exp(-wv / (sig * sigt_ref[...])), 0.0)
        return 0
    lax.fori_loop(0, NB, wexp_blk, 0)

    def w_blk(bi, _):
        sl = pl.ds(bi * BR, BR)
        def sub(bj, _):
            sc = pl.ds(bj * BR, BR)
            t = lax.dot_general(wv_ref[sc, sl], eye_b, (((0,), (0,)), ((), ())),
                                preferred_element_type=jnp.float32,
                                precision=lax.Precision.HIGHEST)
            dist_ref[sl, sc] = (wv_ref[sl, sc] + t) / 2.0
            return 0
        lax.fori_loop(0, NB, sub, 0)
        rs = jnp.sum(dist_ref[sl, :], axis=1, keepdims=True)
        d_inv = 1.0 / (rs + 1e-8)
        coef_ref[sl, :] = lam_ref[sl, :] * d_inv
        return 0
    lax.fori_loop(0, NB, w_blk, 0)

    neg_a = -a_ref[...]
    coef = coef_ref[...]

    def lp_cond(c):
        i, stop = c
        return (i < 50) & jnp.logical_not(stop)

    def lp_body(c):
        i, stop = c
        y_old = y_ref[...]
        z = lax.dot_general(dist_ref[...], y_old, (((1,), (0,)), ((), ())),
                            preferred_element_type=jnp.float32)
        logits = neg_a + coef * z
        mxl = jnp.max(logits, axis=1, keepdims=True)
        el = jnp.exp(logits - mxl)
        y_new = el / jnp.sum(el, axis=1, keepdims=True)
        converged = jnp.max(jnp.abs(y_new - y_old)) < 1e-4

        @pl.when(jnp.logical_not(converged))
        def _():
            y_ref[...] = y_new
        return i + 1, converged

    lax.while_loop(lp_cond, lp_body, (jnp.int32(0), jnp.bool_(False)))

    y = y_ref[...]
    mxy = jnp.max(y, axis=1, keepdims=True)
    cols = lax.broadcasted_iota(jnp.int32, (N, C), 1)
    out_ref[...] = jnp.min(jnp.where(y == mxy, cols, C), axis=1, keepdims=True)


def kernel(feat_s, y_s, feat_q):
    ys2d = y_s.astype(jnp.int32).reshape(1, NS)
    out = pl.pallas_call(
        _body,
        out_shape=jax.ShapeDtypeStruct((N, 1), jnp.int32),
        scratch_shapes=[
            pltpu.VMEM((N, N), jnp.float32),
            pltpu.VMEM((N, N), jnp.float32),
            pltpu.VMEM((N, 1), jnp.float32),
            pltpu.VMEM((1, N), jnp.float32),
            pltpu.VMEM((N, 1), jnp.float32),
            pltpu.VMEM((1, N), jnp.float32),
            pltpu.VMEM((N, C), jnp.float32),
            pltpu.VMEM((N, 1), jnp.float32),
            pltpu.VMEM((N, C), jnp.float32),
            pltpu.VMEM((N, 1), jnp.float32),
        ],
    )(feat_s, ys2d, feat_q)
    return out.reshape(N)


# speedup vs baseline: 11.6142x; 1.1346x over previous
import jax
import jax.numpy as jnp
from jax import lax
from jax.experimental import pallas as pl
from jax.experimental.pallas import tpu as pltpu

N = 2048
DIM = 512
NS = 256
C = 16
KP1 = 12
BR = 256
NB = N // BR
BIGF = 3.0e38


def _body(fs_ref, ys_ref, fq_ref, out_ref,
          dist_ref, wv_ref, q2_ref, q2t_ref, sig_ref, sigt_ref,
          a_ref, lam_ref, y_ref, coef_ref):
    eye_b = (lax.broadcasted_iota(jnp.int32, (BR, BR), 0)
             == lax.broadcasted_iota(jnp.int32, (BR, BR), 1)).astype(jnp.float32)

    def col2row(col_ref, row_ref):
        def blk(bi, _):
            row_ref[:, pl.ds(bi * BR, BR)] = lax.dot_general(
                col_ref[pl.ds(bi * BR, BR), :], eye_b, (((0,), (0,)), ((), ())),
                preferred_element_type=jnp.float32, precision=lax.Precision.HIGHEST)
            return 0
        lax.fori_loop(0, NB, blk, 0)

    def q2_blk(bi, _):
        fq = fq_ref[pl.ds(bi * BR, BR), :]
        q2_ref[pl.ds(bi * BR, BR), :] = jnp.sum(fq * fq, axis=1, keepdims=True)
        wv_ref[pl.ds(bi * BR, BR), :] = jnp.zeros((BR, N), jnp.float32)
        return 0
    lax.fori_loop(0, NB, q2_blk, 0)
    col2row(q2_ref, q2t_ref)

    cls_iota = lax.broadcasted_iota(jnp.int32, (C, NS), 0)
    onehot = (cls_iota == ys_ref[...]).astype(jnp.float32)
    counts = jnp.sum(onehot, axis=1, keepdims=True)
    protos = lax.dot_general(
        onehot, fs_ref[...], (((1,), (0,)), ((), ())),
        preferred_element_type=jnp.float32,
        precision=lax.Precision.HIGHEST) / counts

    p2 = jnp.sum(protos * protos, axis=1, keepdims=True)
    eye16 = (lax.broadcasted_iota(jnp.int32, (C, C), 0)
             == lax.broadcasted_iota(jnp.int32, (C, C), 1)).astype(jnp.float32)
    p2t = lax.dot_general(p2, eye16, (((0,), (0,)), ((), ())),
                          preferred_element_type=jnp.float32,
                          precision=lax.Precision.HIGHEST)
    pq = lax.dot_general(fq_ref[...], protos, (((1,), (1,)), ((), ())),
                         preferred_element_type=jnp.float32)
    d2p = jnp.maximum(q2_ref[...] + p2t - 2.0 * pq, 0.0)
    distp = jnp.sqrt(d2p + 1e-12)
    a_ref[...] = distp * distp
    d = jnp.min(distp, axis=1, keepdims=True)

    db = lax.bitcast_convert_type(d, jnp.int32)
    def med_step(_, c):
        lo, hi = c
        mid = lo + (hi - lo) // 2
        cnt = jnp.sum((db <= mid).astype(jnp.int32))
        take = cnt >= (N // 2)
        return jnp.where(take, lo, mid + 1), jnp.where(take, mid, hi)
    lo, _hi = lax.fori_loop(0, 31, med_step, (jnp.int32(0), jnp.int32(2**31 - 1)))
    med = lax.bitcast_convert_type(lo, jnp.float32)
    lam_ref[...] = jnp.exp(-(d * d) / (2.0 * med * med + 1e-8))

    na = -a_ref[...]
    mx = jnp.max(na, axis=1, keepdims=True)
    e = jnp.exp(na - mx)
    y_ref[...] = e / jnp.sum(e, axis=1, keepdims=True)

    def dist_blk(bi, _):
        fq = fq_ref[pl.ds(bi * BR, BR), :]
        g = lax.dot_general(fq, fq_ref[...], (((1,), (1,)), ((), ())),
                            preferred_element_type=jnp.float32)
        d2 = jnp.maximum(q2_ref[pl.ds(bi * BR, BR), :] + q2t_ref[...] - 2.0 * g, 0.0)
        dist_ref[pl.ds(bi * BR, BR), :] = jnp.sqrt(d2 + 1e-12)
        return 0
    lax.fori_loop(0, NB, dist_blk, 0)

    def ext_step(t, _):
        j = t // NB
        bi = t % NB
        sl = pl.ds(bi * BR, BR)
        blk = dist_ref[sl, :]
        m = jnp.min(blk, axis=1, keepdims=True)
        cols = lax.broadcasted_iota(jnp.int32, (BR, N), 1)
        am = jnp.min(jnp.where(blk == m, cols, N), axis=1, keepdims=True)
        chosen = cols == am
        dist_ref[sl, :] = jnp.where(chosen, BIGF, blk)

        @pl.when(j > 0)
        def _():
            wv_ref[sl, :] = jnp.where(chosen, m, wv_ref[sl, :])

        @pl.when(j == KP1 - 1)
        def _():
            sig_ref[sl, :] = m + 1e-8
        return 0
    lax.fori_loop(0, KP1 * NB, ext_step, 0)

    col2row(sig_ref, sigt_ref)

    def wexp_blk(bi, _):
        sl = pl.ds(bi * BR, BR)
        wv = wv_ref[sl, :]
        sig = sig_ref[sl, :]
        wv_ref[sl, :] = jnp.where(wv > 0.0,
                                  jnp.exp(-wv / (sig * sigt_ref[...])), 0.0)
        return 0
    lax.fori_loop(0, NB, wexp_blk, 0)

    def w_blk(bi, _):
        sl = pl.ds(bi * BR, BR)
        def sub(bj, _):
            sc = pl.ds(bj * BR, BR)
            t = jnp.transpose(wv_ref[sc, sl])
            dist_ref[sl, sc] = (wv_ref[sl, sc] + t) / 2.0
            return 0
        lax.fori_loop(0, NB, sub, 0)
        rs = jnp.sum(dist_ref[sl, :], axis=1, keepdims=True)
        d_inv = 1.0 / (rs + 1e-8)
        coef_ref[sl, :] = lam_ref[sl, :] * d_inv
        return 0
    lax.fori_loop(0, NB, w_blk, 0)

    neg_a = -a_ref[...]
    coef = coef_ref[...]

    def lp_cond(c):
        i, stop = c
        return (i < 50) & jnp.logical_not(stop)

    def lp_body(c):
        i, stop = c
        y_old = y_ref[...]
        z = lax.dot_general(dist_ref[...], y_old, (((1,), (0,)), ((), ())),
                            preferred_element_type=jnp.float32)
        logits = neg_a + coef * z
        mxl = jnp.max(logits, axis=1, keepdims=True)
        el = jnp.exp(logits - mxl)
        y_new = el / jnp.sum(el, axis=1, keepdims=True)
        converged = jnp.max(jnp.abs(y_new - y_old)) < 1e-4

        @pl.when(jnp.logical_not(converged))
        def _():
            y_ref[...] = y_new
        return i + 1, converged

    lax.while_loop(lp_cond, lp_body, (jnp.int32(0), jnp.bool_(False)))

    y = y_ref[...]
    mxy = jnp.max(y, axis=1, keepdims=True)
    cols = lax.broadcasted_iota(jnp.int32, (N, C), 1)
    out_ref[...] = jnp.min(jnp.where(y == mxy, cols, C), axis=1, keepdims=True)


def kernel(feat_s, y_s, feat_q):
    ys2d = y_s.astype(jnp.int32).reshape(1, NS)
    out = pl.pallas_call(
        _body,
        out_shape=jax.ShapeDtypeStruct((N, 1), jnp.int32),
        scratch_shapes=[
            pltpu.VMEM((N, N), jnp.float32),
            pltpu.VMEM((N, N), jnp.float32),
            pltpu.VMEM((N, 1), jnp.float32),
            pltpu.VMEM((1, N), jnp.float32),
            pltpu.VMEM((N, 1), jnp.float32),
            pltpu.VMEM((1, N), jnp.float32),
            pltpu.VMEM((N, C), jnp.float32),
            pltpu.VMEM((N, 1), jnp.float32),
            pltpu.VMEM((N, C), jnp.float32),
            pltpu.VMEM((N, 1), jnp.float32),
        ],
    )(feat_s, ys2d, feat_q)
    return out.reshape(N)
